# baseline (speedup 1.0000x reference)
import functools

import jax
import jax.numpy as jnp
from jax import lax
from jax.experimental import pallas as pl
from jax.experimental.pallas import tpu as pltpu
from jax.experimental.pallas import tpu_sc as plsc

N = 10000
F_IN = 128
H = 4
C = 32
HC = H * C
E = 320000
NEG = 0.2
NP = 10240
NTILES = 32
EPT = E // NTILES
K2_CHUNK = 400
K2_NCH = EPT // K2_CHUNK
K3_CHUNK = 80
K3_NCH = EPT // K3_CHUNK
ROWBLK = 1000
GRID1 = N // ROWBLK


def _k1_body(x_ref, w_ref, asf_ref, adf_ref, ssum_ref, h_ref, as_ref,
             ad_ref, m_ref):
    i = pl.program_id(0)
    hb = jnp.dot(x_ref[...], w_ref[...], preferred_element_type=jnp.float32)
    h_ref[...] = hb
    asb = jnp.dot(hb * asf_ref[...], ssum_ref[...],
                  preferred_element_type=jnp.float32)
    adb = jnp.dot(hb * adf_ref[...], ssum_ref[...],
                  preferred_element_type=jnp.float32)
    as_ref[...] = asb
    ad_ref[...] = adb
    pm = jnp.concatenate([jnp.max(asb, axis=0, keepdims=True),
                          jnp.max(adb, axis=0, keepdims=True)], axis=0)

    @pl.when(i == 0)
    def _():
        m_ref[...] = pm

    @pl.when(i > 0)
    def _():
        m_ref[...] = jnp.maximum(m_ref[...], pm)


def _k1(x, W, asf, adf, ssum):
    return pl.pallas_call(
        _k1_body,
        grid=(GRID1,),
        in_specs=[
            pl.BlockSpec((ROWBLK, F_IN), lambda i: (i, 0)),
            pl.BlockSpec((F_IN, HC), lambda i: (0, 0)),
            pl.BlockSpec((1, HC), lambda i: (0, 0)),
            pl.BlockSpec((1, HC), lambda i: (0, 0)),
            pl.BlockSpec((HC, H), lambda i: (0, 0)),
        ],
        out_specs=[
            pl.BlockSpec((ROWBLK, HC), lambda i: (i, 0)),
            pl.BlockSpec((ROWBLK, H), lambda i: (i, 0)),
            pl.BlockSpec((ROWBLK, H), lambda i: (i, 0)),
            pl.BlockSpec((2, H), lambda i: (0, 0)),
        ],
        out_shape=[
            jax.ShapeDtypeStruct((N, HC), jnp.float32),
            jax.ShapeDtypeStruct((N, H), jnp.float32),
            jax.ShapeDtypeStruct((N, H), jnp.float32),
            jax.ShapeDtypeStruct((2, H), jnp.float32),
        ],
    )(x, W, asf, adf, ssum)


def _k2_body(asrc_hbm, adst_hbm, src_hbm, dst_hbm, m_hbm, w_hbm, accs_hbm,
             asrc_v, adst_v, acc_v, src_v0, src_v1, dst_v0, dst_v1,
             w_v0, w_v1, m_v, tsem, lsem0, lsem1, wsem0, wsem1):
    cid = lax.axis_index("c")
    sid = lax.axis_index("s")
    wid = sid * 2 + cid
    pltpu.async_copy(asrc_hbm, asrc_v, tsem)
    pltpu.async_copy(adst_hbm, adst_v, tsem)
    pltpu.async_copy(m_hbm, m_v, tsem)

    lane = lax.iota(jnp.int32, 16)
    zero16 = jnp.zeros((16,), jnp.float32)
    nconst = jnp.full((16,), N, jnp.int32)

    def zrow(r, _):
        acc_v[pl.ds(r * 16, 16)] = zero16
        return 0
    lax.fori_loop(0, (H * NP) // 16, zrow, 0)

    pltpu.make_async_copy(asrc_hbm, asrc_v, tsem).wait()
    pltpu.make_async_copy(adst_hbm, adst_v, tsem).wait()
    pltpu.make_async_copy(m_hbm, m_v, tsem).wait()
    mvec = m_v[...]

    base0 = wid * EPT
    bufs = ((src_v0, dst_v0, w_v0, lsem0, wsem0),
            (src_v1, dst_v1, w_v1, lsem1, wsem1))

    def process(c, sv, dv, wref, lsm, wsm):
        base = base0 + c * K2_CHUNK
        pltpu.make_async_copy(src_hbm.at[pl.ds(base, K2_CHUNK)],
                              sv, lsm).wait()
        pltpu.make_async_copy(dst_hbm.at[pl.ds(base, K2_CHUNK)],
                              dv, lsm).wait()

        @pl.when(c >= 2)
        def _():
            pbase = (base - 2 * K2_CHUNK) * H
            pltpu.make_async_copy(
                wref, w_hbm.at[pl.ds(pbase, K2_CHUNK * H)], wsm).wait()

        def grp(g, _):
            s16 = sv[pl.ds(g * 16, 16)]
            d16 = dv[pl.ds(g * 16, 16)]
            md = jnp.where(s16 == d16, nconst, d16)
            for hh in range(H):
                a1 = plsc.load_gather(asrc_v, [s16 + hh * N])
                a2 = plsc.load_gather(adst_v, [md + hh * NP])
                z = a1 + a2
                z = jnp.maximum(z, NEG * z)
                wv = jnp.exp(z - mvec[hh])
                plsc.addupdate_scatter(acc_v, [md + hh * NP], wv)
                plsc.store_scatter(wref, [(g * 16 + lane) * H + hh], wv)
            return 0
        lax.fori_loop(0, K2_CHUNK // 16, grp, 0)
        pltpu.async_copy(wref, w_hbm.at[pl.ds(base * H, K2_CHUNK * H)], wsm)

    pltpu.async_copy(src_hbm.at[pl.ds(base0, K2_CHUNK)], src_v0, lsem0)
    pltpu.async_copy(dst_hbm.at[pl.ds(base0, K2_CHUNK)], dst_v0, lsem0)

    def pairloop(ph, _):
        for b in range(2):
            c = ph * 2 + b
            sv, dv, wref, lsm, wsm = bufs[b]
            nsv, ndv, _, nlsm, _ = bufs[1 - b]
            nbase = base0 + (c + 1) * K2_CHUNK
            pltpu.async_copy(src_hbm.at[pl.ds(nbase, K2_CHUNK)], nsv, nlsm)
            pltpu.async_copy(dst_hbm.at[pl.ds(nbase, K2_CHUNK)], ndv, nlsm)
            process(c, sv, dv, wref, lsm, wsm)
        return 0
    lax.fori_loop(0, (K2_NCH - 1) // 2, pairloop, 0)
    process(K2_NCH - 1, src_v0, dst_v0, w_v0, lsem0, wsem0)

    lb = base0 + (K2_NCH - 1) * K2_CHUNK
    pltpu.make_async_copy(
        w_v0, w_hbm.at[pl.ds(lb * H, K2_CHUNK * H)], wsem0).wait()
    pltpu.make_async_copy(
        w_v1, w_hbm.at[pl.ds((lb - K2_CHUNK) * H, K2_CHUNK * H)],
        wsem1).wait()

    for hh in range(H):
        pltpu.sync_copy(acc_v.at[pl.ds(hh * NP, NP)],
                        accs_hbm.at[wid].at[hh])


def _k2(a_src, adst_pad, src, dst, m16):
    mesh = plsc.VectorSubcoreMesh(core_axis_name="c", subcore_axis_name="s")
    f = pl.kernel(
        _k2_body,
        compiler_params=pltpu.CompilerParams(needs_layout_passes=False),
        out_type=[
            jax.ShapeDtypeStruct((E * H,), jnp.float32),
            jax.ShapeDtypeStruct((NTILES, H, NP), jnp.float32),
        ],
        mesh=mesh,
        scratch_types=[
            pltpu.VMEM((H * N,), jnp.float32),
            pltpu.VMEM((H * NP,), jnp.float32),
            pltpu.VMEM((H * NP,), jnp.float32),
            pltpu.VMEM((K2_CHUNK,), jnp.int32),
            pltpu.VMEM((K2_CHUNK,), jnp.int32),
            pltpu.VMEM((K2_CHUNK,), jnp.int32),
            pltpu.VMEM((K2_CHUNK,), jnp.int32),
            pltpu.VMEM((K2_CHUNK * H,), jnp.float32),
            pltpu.VMEM((K2_CHUNK * H,), jnp.float32),
            pltpu.VMEM((16,), jnp.float32),
            pltpu.SemaphoreType.DMA,
            pltpu.SemaphoreType.DMA,
            pltpu.SemaphoreType.DMA,
            pltpu.SemaphoreType.DMA,
            pltpu.SemaphoreType.DMA,
        ],
    )
    return f(a_src, adst_pad, src, dst, m16)


def _kinv_body(accs_ref, as_ref, ad_ref, m_ref, inv_ref, lc_ref):
    tot = jnp.sum(accs_ref[...], axis=0)
    m = m_ref[:, 0:1]
    z = as_ref[...] + ad_ref[...]
    z = jnp.maximum(z, NEG * z)
    w = jnp.exp(z - m)
    t0 = tot[:, :N] + w
    inv0 = 1.0 / (t0 + 1e-16)
    invr = 1.0 / (tot[:, N:] + 1e-16)
    inv_ref[...] = jnp.concatenate([inv0, invr], axis=1)
    lc_ref[...] = w * inv0


def _kinv(accs, a_srcT, a_dstT, mcol):
    return pl.pallas_call(
        _kinv_body,
        out_shape=[
            jax.ShapeDtypeStruct((H, NP), jnp.float32),
            jax.ShapeDtypeStruct((H, N), jnp.float32),
        ],
    )(accs, a_srcT, a_dstT, mcol)


def _k3_body(src_hbm, dst_hbm, w_hbm, inv_hbm, h_hbm, accs_hbm,
             src_v0, src_v1, src_v2, dst_v0, dst_v1, dst_v2,
             md_v0, md_v1, md_v2, mdh_v0, mdh_v1, mdh_v2,
             w_v0, w_v1, w_v2, iv_v0, iv_v1, iv_v2,
             rows_v0, rows_v1, rows_v2,
             lsem0, lsem1, lsem2, isem0, isem1, isem2,
             gsem0, gsem1, gsem2, ssem0, ssem1, ssem2, acc_sh):
    cid = lax.axis_index("c")
    sid = lax.axis_index("s")
    wid = sid * 2 + cid

    nconst = jnp.full((16,), N, jnp.int32)
    zero16 = jnp.zeros((16,), jnp.float32)
    lane = lax.iota(jnp.int32, 16)

    def zr(e, _):
        for k in range(HC // 16):
            rows_v0[e, pl.ds(k * 16, 16)] = zero16
        return 0
    lax.fori_loop(0, K3_CHUNK, zr, 0)
    stripe = NP // 16
    r0 = sid * stripe
    for j in range(stripe // K3_CHUNK):
        pltpu.sync_copy(rows_v0, acc_sh.at[pl.ds(r0 + j * K3_CHUNK,
                                                 K3_CHUNK)])
    plsc.subcore_barrier()

    base0 = wid * EPT
    bufs = ((src_v0, dst_v0, md_v0, mdh_v0, w_v0, iv_v0, rows_v0,
             lsem0, isem0, gsem0, ssem0),
            (src_v1, dst_v1, md_v1, mdh_v1, w_v1, iv_v1, rows_v1,
             lsem1, isem1, gsem1, ssem1),
            (src_v2, dst_v2, md_v2, mdh_v2, w_v2, iv_v2, rows_v2,
             lsem2, isem2, gsem2, ssem2))

    def issue_loads(c, B):
        sv, dv, _, _, wv, _, _, lsm = B[0], B[1], B[2], B[3], B[4], B[5], B[6], B[7]
        base = base0 + c * K3_CHUNK
        pltpu.async_copy(src_hbm.at[pl.ds(base, K3_CHUNK)], sv, lsm)
        pltpu.async_copy(dst_hbm.at[pl.ds(base, K3_CHUNK)], dv, lsm)
        pltpu.async_copy(w_hbm.at[pl.ds(base * H, K3_CHUNK * H)], wv, lsm)

    def wait_loads(c, B):
        sv, dv, wv, lsm = B[0], B[1], B[4], B[7]
        base = base0 + c * K3_CHUNK
        pltpu.make_async_copy(src_hbm.at[pl.ds(base, K3_CHUNK)],
                              sv, lsm).wait()
        pltpu.make_async_copy(dst_hbm.at[pl.ds(base, K3_CHUNK)],
                              dv, lsm).wait()
        pltpu.make_async_copy(w_hbm.at[pl.ds(base * H, K3_CHUNK * H)],
                              wv, lsm).wait()

    def issue_gathers(B):
        sv, dv, mdv, mdhv, ivv, rv, ism, gsm = (
            B[0], B[1], B[2], B[3], B[5], B[6], B[8], B[9])

        def grp_idx(g, _):
            s16 = sv[pl.ds(g * 16, 16)]
            d16 = dv[pl.ds(g * 16, 16)]
            md = jnp.where(s16 == d16, nconst, d16)
            mdv[pl.ds(g * 16, 16)] = md
            for hh in range(H):
                mdhv[pl.ds(hh * K3_CHUNK + g * 16, 16)] = md + hh * NP
            return 0
        lax.fori_loop(0, K3_CHUNK // 16, grp_idx, 0)
        pltpu.async_copy(inv_hbm.at[mdhv], ivv, ism)
        pltpu.async_copy(h_hbm.at[sv], rv, gsm)

    def process(B, scatter_sync):
        sv, mdv, mdhv, wv, ivv, rv, ism, gsm, ssm = (
            B[0], B[2], B[3], B[4], B[5], B[6], B[8], B[9], B[10])
        pltpu.make_async_copy(inv_hbm.at[mdhv], ivv, ism).wait()
        pltpu.make_async_copy(h_hbm.at[sv], rv, gsm).wait()

        def grp(g, _):
            for hh in range(H):
                wvec = plsc.load_gather(wv, [(g * 16 + lane) * H + hh])
                cvec = wvec * ivv[pl.ds(hh * K3_CHUNK + g * 16, 16)]
                for e in range(16):
                    cs = cvec[e]
                    for k in range(C // 16):
                        off = hh * C + k * 16
                        rv[g * 16 + e, pl.ds(off, 16)] = (
                            rv[g * 16 + e, pl.ds(off, 16)] * cs)
            return 0
        lax.fori_loop(0, K3_CHUNK // 16, grp, 0)

        if scatter_sync:
            pltpu.sync_copy(rv, acc_sh.at[mdv], add=True)
        else:
            pltpu.async_copy(rv, acc_sh.at[mdv], ssm, add=True)

    def wait_scatter(B):
        pltpu.make_async_copy(B[6], acc_sh.at[B[2]], B[10]).wait()

    issue_loads(0, bufs[0])
    wait_loads(0, bufs[0])
    issue_gathers(bufs[0])
    issue_loads(1, bufs[1])

    def step(c, s):
        B = bufs[s]
        NB = bufs[(s + 1) % 3]
        PB = bufs[(s + 2) % 3]
        wait_loads(c + 1, NB)

        @pl.when(c >= 2)
        def _():
            wait_scatter(NB)
        issue_gathers(NB)
        process(B, scatter_sync=False)
        issue_loads(c + 2, PB)

    def triloop(t, _):
        for j in range(3):
            step(t * 3 + j, j)
        return 0
    nt = (K3_NCH - 2) // 3
    lax.fori_loop(0, nt, triloop, 0)

    c = nt * 3
    wait_loads(c + 1, bufs[1])
    wait_scatter(bufs[1])
    issue_gathers(bufs[1])
    process(bufs[0], scatter_sync=False)
    wait_scatter(bufs[2])
    process(bufs[1], scatter_sync=True)
    wait_scatter(bufs[0])

    plsc.subcore_barrier()
    pltpu.sync_copy(acc_sh.at[pl.ds(r0, stripe)],
                    accs_hbm.at[cid].at[pl.ds(r0, stripe)])


def _k3(src, dst, w4e, invflat, h):
    mesh = plsc.VectorSubcoreMesh(core_axis_name="c", subcore_axis_name="s")
    f = pl.kernel(
        _k3_body,
        compiler_params=pltpu.CompilerParams(needs_layout_passes=False),
        out_type=[jax.ShapeDtypeStruct((2, NP, HC), jnp.float32)],
        mesh=mesh,
        scratch_types=(
            [pltpu.VMEM((K3_CHUNK,), jnp.int32)] * 3
            + [pltpu.VMEM((K3_CHUNK,), jnp.int32)] * 3
            + [pltpu.VMEM((K3_CHUNK,), jnp.int32)] * 3
            + [pltpu.VMEM((H * K3_CHUNK,), jnp.int32)] * 3
            + [pltpu.VMEM((H * K3_CHUNK,), jnp.float32)] * 3
            + [pltpu.VMEM((H * K3_CHUNK,), jnp.float32)] * 3
            + [pltpu.VMEM((K3_CHUNK, HC), jnp.float32)] * 3
            + [pltpu.SemaphoreType.DMA] * 12
            + [pltpu.VMEM_SHARED((NP, HC), jnp.float32)]
        ),
    )
    return f(src, dst, w4e, invflat, h)[0]


def _k4_body(acc_ref, h_ref, lc_ref, sexp_ref, b_ref, o_ref):
    lcx = jnp.dot(lc_ref[...], sexp_ref[...],
                  preferred_element_type=jnp.float32)
    o_ref[...] = (acc_ref[0] + acc_ref[1] + h_ref[...] * lcx + b_ref[...])


def _k4(accs2, h, lc, sexp, bias):
    return pl.pallas_call(
        _k4_body,
        grid=(GRID1,),
        in_specs=[
            pl.BlockSpec((2, ROWBLK, HC), lambda i: (0, i, 0)),
            pl.BlockSpec((ROWBLK, HC), lambda i: (i, 0)),
            pl.BlockSpec((ROWBLK, H), lambda i: (i, 0)),
            pl.BlockSpec((H, HC), lambda i: (0, 0)),
            pl.BlockSpec((1, HC), lambda i: (0, 0)),
        ],
        out_specs=pl.BlockSpec((ROWBLK, HC), lambda i: (i, 0)),
        out_shape=jax.ShapeDtypeStruct((N, HC), jnp.float32),
    )(accs2, h, lc, sexp, bias)


@jax.jit
def kernel(x, edge_index, W, att_src, att_dst, bias):
    src = edge_index[0]
    dst = edge_index[1]
    asf = att_src.reshape(1, HC)
    adf = att_dst.reshape(1, HC)
    ssum = jnp.repeat(jnp.eye(H, dtype=jnp.float32), C, axis=0)

    h, a_src, a_dst, m2 = _k1(x, W, asf, adf, ssum)

    m4 = jnp.maximum(m2[0] + m2[1], 0.0)
    m16 = jnp.pad(m4, (0, 16 - H))
    mcol = jnp.broadcast_to(m4.reshape(H, 1), (H, HC))
    a_srcT = a_src.T
    a_dstT = a_dst.T
    adstT_pad = jnp.zeros((H, NP), jnp.float32).at[:, :N].set(a_dstT)

    w4e, accs1 = _k2(a_srcT.reshape(-1), adstT_pad.reshape(-1),
                     src, dst, m16)
    inv, lc = _kinv(accs1, a_srcT, a_dstT, mcol)
    accs2 = _k3(src, dst, w4e, inv.reshape(-1), h)
    return _k4(accs2, h, lc.T, ssum.T, bias.reshape(1, HC))


# speedup vs baseline: 1.0641x; 1.0641x over previous
import functools

import jax
import jax.numpy as jnp
from jax import lax
from jax.experimental import pallas as pl
from jax.experimental.pallas import tpu as pltpu
from jax.experimental.pallas import tpu_sc as plsc

N = 10000
F_IN = 128
H = 4
C = 32
HC = H * C
E = 320000
NEG = 0.2
NP = 10240
NTILES = 32
EPT = E // NTILES
K2_CHUNK = 400
K2_NCH = EPT // K2_CHUNK
K3_CHUNK = 80
K3_NCH = EPT // K3_CHUNK
ROWBLK = 1000
GRID1 = N // ROWBLK


def _k1_body(x_ref, w_ref, asf_ref, adf_ref, ssum_ref, h_ref, as_ref,
             ad_ref, m_ref):
    i = pl.program_id(0)
    hb = jnp.dot(x_ref[...], w_ref[...], preferred_element_type=jnp.float32)
    h_ref[...] = hb
    asb = jnp.dot(hb * asf_ref[...], ssum_ref[...],
                  preferred_element_type=jnp.float32)
    adb = jnp.dot(hb * adf_ref[...], ssum_ref[...],
                  preferred_element_type=jnp.float32)
    as_ref[...] = asb
    ad_ref[...] = adb
    pm = jnp.concatenate([jnp.max(asb, axis=0, keepdims=True),
                          jnp.max(adb, axis=0, keepdims=True)], axis=0)

    @pl.when(i == 0)
    def _():
        m_ref[...] = pm

    @pl.when(i > 0)
    def _():
        m_ref[...] = jnp.maximum(m_ref[...], pm)


def _k1(x, W, asf, adf, ssum):
    return pl.pallas_call(
        _k1_body,
        grid=(GRID1,),
        in_specs=[
            pl.BlockSpec((ROWBLK, F_IN), lambda i: (i, 0)),
            pl.BlockSpec((F_IN, HC), lambda i: (0, 0)),
            pl.BlockSpec((1, HC), lambda i: (0, 0)),
            pl.BlockSpec((1, HC), lambda i: (0, 0)),
            pl.BlockSpec((HC, H), lambda i: (0, 0)),
        ],
        out_specs=[
            pl.BlockSpec((ROWBLK, HC), lambda i: (i, 0)),
            pl.BlockSpec((ROWBLK, H), lambda i: (i, 0)),
            pl.BlockSpec((ROWBLK, H), lambda i: (i, 0)),
            pl.BlockSpec((2, H), lambda i: (0, 0)),
        ],
        out_shape=[
            jax.ShapeDtypeStruct((N, HC), jnp.float32),
            jax.ShapeDtypeStruct((N, H), jnp.float32),
            jax.ShapeDtypeStruct((N, H), jnp.float32),
            jax.ShapeDtypeStruct((2, H), jnp.float32),
        ],
    )(x, W, asf, adf, ssum)


def _k2_body(asrc_hbm, adst_hbm, src_hbm, dst_hbm, m_hbm, w_hbm, accs_hbm,
             asrc_v, adst_v, acc_v, src_v0, src_v1, dst_v0, dst_v1,
             w_v0, w_v1, m_v, tsem, lsem0, lsem1, wsem0, wsem1):
    cid = lax.axis_index("c")
    sid = lax.axis_index("s")
    wid = sid * 2 + cid
    pltpu.async_copy(asrc_hbm, asrc_v, tsem)
    pltpu.async_copy(adst_hbm, adst_v, tsem)
    pltpu.async_copy(m_hbm, m_v, tsem)

    lane = lax.iota(jnp.int32, 16)
    zero16 = jnp.zeros((16,), jnp.float32)
    nconst = jnp.full((16,), N, jnp.int32)

    def zrow(r, _):
        acc_v[pl.ds(r * 16, 16)] = zero16
        return 0
    lax.fori_loop(0, (H * NP) // 16, zrow, 0)

    pltpu.make_async_copy(asrc_hbm, asrc_v, tsem).wait()
    pltpu.make_async_copy(adst_hbm, adst_v, tsem).wait()
    pltpu.make_async_copy(m_hbm, m_v, tsem).wait()
    mvec = m_v[...]

    base0 = wid * EPT
    bufs = ((src_v0, dst_v0, w_v0, lsem0, wsem0),
            (src_v1, dst_v1, w_v1, lsem1, wsem1))

    def process(c, sv, dv, wref, lsm, wsm):
        base = base0 + c * K2_CHUNK
        pltpu.make_async_copy(src_hbm.at[pl.ds(base, K2_CHUNK)],
                              sv, lsm).wait()
        pltpu.make_async_copy(dst_hbm.at[pl.ds(base, K2_CHUNK)],
                              dv, lsm).wait()

        @pl.when(c >= 2)
        def _():
            pbase = (base - 2 * K2_CHUNK) * H
            pltpu.make_async_copy(
                wref, w_hbm.at[pl.ds(pbase, K2_CHUNK * H)], wsm).wait()

        def grp(g, _):
            s16 = sv[pl.ds(g * 16, 16)]
            d16 = dv[pl.ds(g * 16, 16)]
            md = jnp.where(s16 == d16, nconst, d16)
            for hh in range(H):
                a1 = plsc.load_gather(asrc_v, [s16 + hh * N])
                a2 = plsc.load_gather(adst_v, [md + hh * NP])
                z = a1 + a2
                z = jnp.maximum(z, NEG * z)
                wv = jnp.exp(z - mvec[hh])
                plsc.addupdate_scatter(acc_v, [md + hh * NP], wv)
                plsc.store_scatter(wref, [(g * 16 + lane) * H + hh], wv)
            return 0
        lax.fori_loop(0, K2_CHUNK // 16, grp, 0)
        pltpu.async_copy(wref, w_hbm.at[pl.ds(base * H, K2_CHUNK * H)], wsm)

    pltpu.async_copy(src_hbm.at[pl.ds(base0, K2_CHUNK)], src_v0, lsem0)
    pltpu.async_copy(dst_hbm.at[pl.ds(base0, K2_CHUNK)], dst_v0, lsem0)

    def pairloop(ph, _):
        for b in range(2):
            c = ph * 2 + b
            sv, dv, wref, lsm, wsm = bufs[b]
            nsv, ndv, _, nlsm, _ = bufs[1 - b]
            nbase = base0 + (c + 1) * K2_CHUNK
            pltpu.async_copy(src_hbm.at[pl.ds(nbase, K2_CHUNK)], nsv, nlsm)
            pltpu.async_copy(dst_hbm.at[pl.ds(nbase, K2_CHUNK)], ndv, nlsm)
            process(c, sv, dv, wref, lsm, wsm)
        return 0
    lax.fori_loop(0, (K2_NCH - 1) // 2, pairloop, 0)
    process(K2_NCH - 1, src_v0, dst_v0, w_v0, lsem0, wsem0)

    lb = base0 + (K2_NCH - 1) * K2_CHUNK
    pltpu.make_async_copy(
        w_v0, w_hbm.at[pl.ds(lb * H, K2_CHUNK * H)], wsem0).wait()
    pltpu.make_async_copy(
        w_v1, w_hbm.at[pl.ds((lb - K2_CHUNK) * H, K2_CHUNK * H)],
        wsem1).wait()

    for hh in range(H):
        pltpu.sync_copy(acc_v.at[pl.ds(hh * NP, NP)],
                        accs_hbm.at[wid].at[hh])


def _k2(a_src, adst_pad, src, dst, m16):
    mesh = plsc.VectorSubcoreMesh(core_axis_name="c", subcore_axis_name="s")
    f = pl.kernel(
        _k2_body,
        compiler_params=pltpu.CompilerParams(needs_layout_passes=False),
        out_type=[
            jax.ShapeDtypeStruct((E * H,), jnp.float32),
            jax.ShapeDtypeStruct((NTILES, H, NP), jnp.float32),
        ],
        mesh=mesh,
        scratch_types=[
            pltpu.VMEM((H * N,), jnp.float32),
            pltpu.VMEM((H * NP,), jnp.float32),
            pltpu.VMEM((H * NP,), jnp.float32),
            pltpu.VMEM((K2_CHUNK,), jnp.int32),
            pltpu.VMEM((K2_CHUNK,), jnp.int32),
            pltpu.VMEM((K2_CHUNK,), jnp.int32),
            pltpu.VMEM((K2_CHUNK,), jnp.int32),
            pltpu.VMEM((K2_CHUNK * H,), jnp.float32),
            pltpu.VMEM((K2_CHUNK * H,), jnp.float32),
            pltpu.VMEM((16,), jnp.float32),
            pltpu.SemaphoreType.DMA,
            pltpu.SemaphoreType.DMA,
            pltpu.SemaphoreType.DMA,
            pltpu.SemaphoreType.DMA,
            pltpu.SemaphoreType.DMA,
        ],
    )
    return f(a_src, adst_pad, src, dst, m16)


def _kinv_body(accs_ref, as_ref, ad_ref, m_ref, inv_ref, lc_ref):
    tot = jnp.sum(accs_ref[...], axis=0)
    m = m_ref[:, 0:1]
    z = as_ref[...] + ad_ref[...]
    z = jnp.maximum(z, NEG * z)
    w = jnp.exp(z - m)
    t0 = tot[:, :N] + w
    inv0 = 1.0 / (t0 + 1e-16)
    inv_ref[...] = inv0.T
    lc_ref[...] = (w * inv0).T


def _kinv(accs, a_srcT, a_dstT, mcol):
    return pl.pallas_call(
        _kinv_body,
        out_shape=[
            jax.ShapeDtypeStruct((N, H), jnp.float32),
            jax.ShapeDtypeStruct((N, H), jnp.float32),
        ],
    )(accs, a_srcT, a_dstT, mcol)


def _k3_body(src_hbm, dst_hbm, w_hbm, h_hbm, accs_hbm,
             src_v0, src_v1, src_v2, dst_v0, dst_v1, dst_v2,
             md_v0, md_v1, md_v2,
             w_v0, w_v1, w_v2,
             rows_v0, rows_v1, rows_v2,
             lsem0, lsem1, lsem2,
             gsem0, gsem1, gsem2, ssem0, ssem1, ssem2, acc_sh):
    cid = lax.axis_index("c")
    sid = lax.axis_index("s")
    wid = sid * 2 + cid

    nconst = jnp.full((16,), N, jnp.int32)
    zero16 = jnp.zeros((16,), jnp.float32)
    lane = lax.iota(jnp.int32, 16)

    def zr(e, _):
        for k in range(HC // 16):
            rows_v0[e, pl.ds(k * 16, 16)] = zero16
        return 0
    lax.fori_loop(0, K3_CHUNK, zr, 0)
    stripe = NP // 16
    r0 = sid * stripe
    for j in range(stripe // K3_CHUNK):
        pltpu.sync_copy(rows_v0, acc_sh.at[pl.ds(r0 + j * K3_CHUNK,
                                                 K3_CHUNK)])
    plsc.subcore_barrier()

    base0 = wid * EPT
    bufs = ((src_v0, dst_v0, md_v0, w_v0, rows_v0, lsem0, gsem0, ssem0),
            (src_v1, dst_v1, md_v1, w_v1, rows_v1, lsem1, gsem1, ssem1),
            (src_v2, dst_v2, md_v2, w_v2, rows_v2, lsem2, gsem2, ssem2))

    def issue_loads(c, B):
        sv, dv, wv, lsm = B[0], B[1], B[3], B[5]
        base = base0 + c * K3_CHUNK
        pltpu.async_copy(src_hbm.at[pl.ds(base, K3_CHUNK)], sv, lsm)
        pltpu.async_copy(dst_hbm.at[pl.ds(base, K3_CHUNK)], dv, lsm)
        pltpu.async_copy(w_hbm.at[pl.ds(base * H, K3_CHUNK * H)], wv, lsm)

    def wait_loads(c, B):
        sv, dv, wv, lsm = B[0], B[1], B[3], B[5]
        base = base0 + c * K3_CHUNK
        pltpu.make_async_copy(src_hbm.at[pl.ds(base, K3_CHUNK)],
                              sv, lsm).wait()
        pltpu.make_async_copy(dst_hbm.at[pl.ds(base, K3_CHUNK)],
                              dv, lsm).wait()
        pltpu.make_async_copy(w_hbm.at[pl.ds(base * H, K3_CHUNK * H)],
                              wv, lsm).wait()

    def issue_gathers(B):
        sv, dv, mdv, rv, gsm = B[0], B[1], B[2], B[4], B[6]

        def grp_idx(g, _):
            s16 = sv[pl.ds(g * 16, 16)]
            d16 = dv[pl.ds(g * 16, 16)]
            md = jnp.where(s16 == d16, nconst, d16)
            mdv[pl.ds(g * 16, 16)] = md
            return 0
        lax.fori_loop(0, K3_CHUNK // 16, grp_idx, 0)
        pltpu.async_copy(h_hbm.at[sv], rv, gsm)

    def process(B, scatter_sync):
        sv, mdv, wv, rv, gsm, ssm = (
            B[0], B[2], B[3], B[4], B[6], B[7])
        pltpu.make_async_copy(h_hbm.at[sv], rv, gsm).wait()

        def grp(g, _):
            e16 = g * 16 + lane
            for hh in range(H):
                cvec = plsc.load_gather(wv, [e16 * H + hh])
                for e in range(16):
                    cs = cvec[e]
                    for k in range(C // 16):
                        off = hh * C + k * 16
                        rv[g * 16 + e, pl.ds(off, 16)] = (
                            rv[g * 16 + e, pl.ds(off, 16)] * cs)
            return 0
        lax.fori_loop(0, K3_CHUNK // 16, grp, 0)

        if scatter_sync:
            pltpu.sync_copy(rv, acc_sh.at[mdv], add=True)
        else:
            pltpu.async_copy(rv, acc_sh.at[mdv], ssm, add=True)

    def wait_scatter(B):
        pltpu.make_async_copy(B[4], acc_sh.at[B[2]], B[7]).wait()

    issue_loads(0, bufs[0])
    wait_loads(0, bufs[0])
    issue_gathers(bufs[0])
    issue_loads(1, bufs[1])

    def step(c, s):
        B = bufs[s]
        NB = bufs[(s + 1) % 3]
        PB = bufs[(s + 2) % 3]
        wait_loads(c + 1, NB)

        @pl.when(c >= 2)
        def _():
            wait_scatter(NB)
        issue_gathers(NB)
        process(B, scatter_sync=False)
        issue_loads(c + 2, PB)

    def triloop(t, _):
        for j in range(3):
            step(t * 3 + j, j)
        return 0
    nt = (K3_NCH - 2) // 3
    lax.fori_loop(0, nt, triloop, 0)

    c = nt * 3
    wait_loads(c + 1, bufs[1])
    wait_scatter(bufs[1])
    issue_gathers(bufs[1])
    process(bufs[0], scatter_sync=False)
    wait_scatter(bufs[2])
    process(bufs[1], scatter_sync=True)
    wait_scatter(bufs[0])

    plsc.subcore_barrier()
    pltpu.sync_copy(acc_sh.at[pl.ds(r0, stripe)],
                    accs_hbm.at[cid].at[pl.ds(r0, stripe)])


def _k3(src, dst, w4e, h):
    mesh = plsc.VectorSubcoreMesh(core_axis_name="c", subcore_axis_name="s")
    f = pl.kernel(
        _k3_body,
        compiler_params=pltpu.CompilerParams(needs_layout_passes=False),
        out_type=[jax.ShapeDtypeStruct((2, NP, HC), jnp.float32)],
        mesh=mesh,
        scratch_types=(
            [pltpu.VMEM((K3_CHUNK,), jnp.int32)] * 3
            + [pltpu.VMEM((K3_CHUNK,), jnp.int32)] * 3
            + [pltpu.VMEM((K3_CHUNK,), jnp.int32)] * 3
            + [pltpu.VMEM((H * K3_CHUNK,), jnp.float32)] * 3
            + [pltpu.VMEM((K3_CHUNK, HC), jnp.float32)] * 3
            + [pltpu.SemaphoreType.DMA] * 9
            + [pltpu.VMEM_SHARED((NP, HC), jnp.float32)]
        ),
    )
    return f(src, dst, w4e, h)[0]


def _k4_body(acc_ref, h_ref, inv_ref, lc_ref, sexp_ref, b_ref, o_ref):
    invx = jnp.dot(inv_ref[...], sexp_ref[...],
                   preferred_element_type=jnp.float32)
    lcx = jnp.dot(lc_ref[...], sexp_ref[...],
                  preferred_element_type=jnp.float32)
    o_ref[...] = ((acc_ref[0] + acc_ref[1]) * invx + h_ref[...] * lcx
                  + b_ref[...])


def _k4(accs2, h, inv, lc, sexp, bias):
    return pl.pallas_call(
        _k4_body,
        grid=(GRID1,),
        in_specs=[
            pl.BlockSpec((2, ROWBLK, HC), lambda i: (0, i, 0)),
            pl.BlockSpec((ROWBLK, HC), lambda i: (i, 0)),
            pl.BlockSpec((ROWBLK, H), lambda i: (i, 0)),
            pl.BlockSpec((ROWBLK, H), lambda i: (i, 0)),
            pl.BlockSpec((H, HC), lambda i: (0, 0)),
            pl.BlockSpec((1, HC), lambda i: (0, 0)),
        ],
        out_specs=pl.BlockSpec((ROWBLK, HC), lambda i: (i, 0)),
        out_shape=jax.ShapeDtypeStruct((N, HC), jnp.float32),
    )(accs2, h, inv, lc, sexp, bias)


@jax.jit
def kernel(x, edge_index, W, att_src, att_dst, bias):
    src = edge_index[0]
    dst = edge_index[1]
    asf = att_src.reshape(1, HC)
    adf = att_dst.reshape(1, HC)
    ssum = jnp.repeat(jnp.eye(H, dtype=jnp.float32), C, axis=0)

    h, a_src, a_dst, m2 = _k1(x, W, asf, adf, ssum)

    m4 = jnp.maximum(m2[0] + m2[1], 0.0)
    m16 = jnp.pad(m4, (0, 16 - H))
    mcol = jnp.broadcast_to(m4.reshape(H, 1), (H, HC))
    a_srcT = a_src.T
    a_dstT = a_dst.T
    adstT_pad = jnp.zeros((H, NP), jnp.float32).at[:, :N].set(a_dstT)

    w4e, accs1 = _k2(a_srcT.reshape(-1), adstT_pad.reshape(-1),
                     src, dst, m16)
    inv, lc = _kinv(accs1, a_srcT, a_dstT, mcol)
    accs2 = _k3(src, dst, w4e, h)
    return _k4(accs2, h, inv, lc, ssum.T, bias.reshape(1, HC))
